# baseline (speedup 1.0000x reference)
import jax
import jax.numpy as jnp
from jax import lax
from jax.experimental import pallas as pl
from jax.experimental.pallas import tpu as pltpu

_N_EXP = 2
_D = 1024
_NA = 16
_TILE = 2048
_SMALL = 128


def _tile_body(instr_ref, x_ref, r_ref, w1_ref, wh_ref, b1_ref,
               wc_ref, bc_ref, h_out_ref, small_out_ref):
    x = x_ref[...]
    r = r_ref[...]
    col = lax.broadcasted_iota(jnp.int32, (_TILE, _SMALL), 1)
    is_logit = col < _NA
    for e in range(_N_EXP):
        pre = (jnp.dot(x, w1_ref[e], preferred_element_type=jnp.float32)
               + jnp.dot(r, wh_ref[e], preferred_element_type=jnp.float32)
               + b1_ref[e][None, :])
        h = jnp.tanh(pre)
        head = (jnp.dot(h, wc_ref[e], preferred_element_type=jnp.float32)
                + bc_ref[e][None, :])
        ml = jnp.where(is_logit, head, jnp.float32(-1e30))
        m = jnp.max(ml, axis=1, keepdims=True)
        amax = jnp.min(jnp.where((ml == m) & is_logit, col, _SMALL),
                       axis=1, keepdims=True).astype(jnp.float32)
        se = jnp.sum(jnp.where(is_logit, jnp.exp(ml - m), 0.0),
                     axis=1, keepdims=True)
        lp = -jnp.log(se)
        v = jnp.sum(jnp.where(col == _NA, head, 0.0), axis=1, keepdims=True)
        small = jnp.where(col == 0, v,
                          jnp.where(col == 1, amax,
                                    jnp.where(col == 2, lp, 0.0)))
        flag = instr_ref[...] == jnp.float32(e)
        if e == 0:
            h_out_ref[...] = jnp.where(flag, h, 0.0)
            small_out_ref[...] = jnp.where(flag, small, 0.0)
        else:
            h_out_ref[...] = jnp.where(flag, h, h_out_ref[...])
            small_out_ref[...] = jnp.where(flag, small, small_out_ref[...])


def kernel(base, instructions, rnn_hxs, masks, W1, b1, Wh, Wv, bv, Wa, ba):
    n, d = base.shape
    rnnm = rnn_hxs * masks
    instrf = instructions.astype(jnp.float32).reshape(n, 1)
    w1 = W1[:_N_EXP]
    wh = Wh[:_N_EXP]
    b1s = b1[:_N_EXP]
    pad = jnp.zeros((_N_EXP, d, _SMALL - _NA - 1), dtype=jnp.float32)
    wc = jnp.concatenate([Wa[:_N_EXP], Wv[:_N_EXP], pad], axis=-1)
    bpad = jnp.zeros((_N_EXP, _SMALL - _NA - 1), dtype=jnp.float32)
    bc = jnp.concatenate([ba[:_N_EXP], bv[:_N_EXP], bpad], axis=-1)

    grid = (n // _TILE,)
    h_out, small_out = pl.pallas_call(
        _tile_body,
        grid=grid,
        in_specs=[
            pl.BlockSpec((_TILE, 1), lambda t: (t, 0)),
            pl.BlockSpec((_TILE, d), lambda t: (t, 0)),
            pl.BlockSpec((_TILE, d), lambda t: (t, 0)),
            pl.BlockSpec((_N_EXP, d, d), lambda t: (0, 0, 0)),
            pl.BlockSpec((_N_EXP, d, d), lambda t: (0, 0, 0)),
            pl.BlockSpec((_N_EXP, d), lambda t: (0, 0)),
            pl.BlockSpec((_N_EXP, d, _SMALL), lambda t: (0, 0, 0)),
            pl.BlockSpec((_N_EXP, _SMALL), lambda t: (0, 0)),
        ],
        out_specs=[
            pl.BlockSpec((_TILE, d), lambda t: (t, 0)),
            pl.BlockSpec((_TILE, _SMALL), lambda t: (t, 0)),
        ],
        out_shape=[
            jax.ShapeDtypeStruct((n, d), jnp.float32),
            jax.ShapeDtypeStruct((n, _SMALL), jnp.float32),
        ],
        compiler_params=pltpu.CompilerParams(
            dimension_semantics=("arbitrary",),
        ),
    )(instrf, base, rnnm, w1, wh, b1s, wc, bc)

    value = small_out[:, 0:1]
    action = small_out[:, 1:2].astype(jnp.int32)
    alp = small_out[:, 2:3]
    return value, action, alp, h_out


# speedup vs baseline: 1.3703x; 1.3703x over previous
import jax
import jax.numpy as jnp
from jax import lax
from jax.experimental import pallas as pl
from jax.experimental.pallas import tpu as pltpu

_N_EXP = 2
_NE_TOT = 3
_D = 1024
_NA = 16
_TILE = 256
_SMALL = 128


def _tile_body(instr_ref, mask_ref, x_ref, r_ref, w1_ref, wh_ref, b1_ref,
               wa_ref, ba_ref, wv_ref, bv_ref, h_out_ref, small_out_ref):
    x = x_ref[...]
    r = r_ref[...] * mask_ref[...]
    col = lax.broadcasted_iota(jnp.int32, (_TILE, _SMALL), 1)
    for e in range(_N_EXP):
        pre = (jnp.dot(x, w1_ref[e], preferred_element_type=jnp.float32)
               + jnp.dot(r, wh_ref[e], preferred_element_type=jnp.float32)
               + b1_ref[e][None, :])
        h = jnp.tanh(pre)
        logits = (jnp.dot(h, wa_ref[e], preferred_element_type=jnp.float32)
                  + ba_ref[e][None, :])
        v = (jnp.dot(h, wv_ref[e], preferred_element_type=jnp.float32)
             + bv_ref[e][None, :])
        m = jnp.max(logits, axis=1, keepdims=True)
        lcol = lax.broadcasted_iota(jnp.int32, (_TILE, _NA), 1)
        amax = jnp.min(jnp.where(logits == m, lcol, _NA),
                       axis=1, keepdims=True).astype(jnp.float32)
        se = jnp.sum(jnp.exp(logits - m), axis=1, keepdims=True)
        lp = -jnp.log(se)
        small = jnp.where(col == 0, v,
                          jnp.where(col == 1, amax,
                                    jnp.where(col == 2, lp, 0.0)))
        flag = instr_ref[...] == jnp.float32(e)
        if e == 0:
            h_out_ref[...] = jnp.where(flag, h, 0.0)
            small_out_ref[...] = jnp.where(flag, small, 0.0)
        else:
            h_out_ref[...] = jnp.where(flag, h, h_out_ref[...])
            small_out_ref[...] = jnp.where(flag, small, small_out_ref[...])


def kernel(base, instructions, rnn_hxs, masks, W1, b1, Wh, Wv, bv, Wa, ba):
    n, d = base.shape
    instrf = instructions.astype(jnp.float32).reshape(n, 1)

    grid = (n // _TILE,)
    h_out, small_out = pl.pallas_call(
        _tile_body,
        grid=grid,
        in_specs=[
            pl.BlockSpec((_TILE, 1), lambda t: (t, 0)),
            pl.BlockSpec((_TILE, 1), lambda t: (t, 0)),
            pl.BlockSpec((_TILE, d), lambda t: (t, 0)),
            pl.BlockSpec((_TILE, d), lambda t: (t, 0)),
            pl.BlockSpec((_N_EXP, d, d), lambda t: (0, 0, 0)),
            pl.BlockSpec((_N_EXP, d, d), lambda t: (0, 0, 0)),
            pl.BlockSpec((_NE_TOT, d), lambda t: (0, 0)),
            pl.BlockSpec((_N_EXP, d, _NA), lambda t: (0, 0, 0)),
            pl.BlockSpec((_NE_TOT, _NA), lambda t: (0, 0)),
            pl.BlockSpec((_N_EXP, d, 1), lambda t: (0, 0, 0)),
            pl.BlockSpec((_NE_TOT, 1), lambda t: (0, 0)),
        ],
        out_specs=[
            pl.BlockSpec((_TILE, d), lambda t: (t, 0)),
            pl.BlockSpec((_TILE, _SMALL), lambda t: (t, 0)),
        ],
        out_shape=[
            jax.ShapeDtypeStruct((n, d), jnp.float32),
            jax.ShapeDtypeStruct((n, _SMALL), jnp.float32),
        ],
        compiler_params=pltpu.CompilerParams(
            dimension_semantics=("arbitrary",),
        ),
    )(instrf, masks, base, rnn_hxs, W1, Wh, b1, Wa, ba, Wv, bv)

    value = small_out[:, 0:1]
    action = small_out[:, 1:2].astype(jnp.int32)
    alp = small_out[:, 2:3]
    return value, action, alp, h_out


# speedup vs baseline: 1.4217x; 1.0376x over previous
import jax
import jax.numpy as jnp
from jax import lax
from jax.experimental import pallas as pl
from jax.experimental.pallas import tpu as pltpu

_N_EXP = 2
_NE_TOT = 3
_D = 1024
_NA = 16
_TILE = 256


def _tile_body(instr_ref, mask_ref, x_ref, r_ref, w1_ref, wh_ref, b1_ref,
               wa_ref, ba_ref, wv_ref, bv_ref,
               v_ref, a_ref, lp_ref, h_ref):
    x = x_ref[...]
    r = r_ref[...] * mask_ref[...]
    for e in range(_N_EXP):
        pre = (jnp.dot(x, w1_ref[e], preferred_element_type=jnp.float32)
               + jnp.dot(r, wh_ref[e], preferred_element_type=jnp.float32)
               + b1_ref[e][None, :])
        h = jnp.tanh(pre)
        logits = (jnp.dot(h, wa_ref[e], preferred_element_type=jnp.float32)
                  + ba_ref[e][None, :])
        v = (jnp.dot(h, wv_ref[e], preferred_element_type=jnp.float32)
             + bv_ref[e][None, :])
        m = jnp.max(logits, axis=1, keepdims=True)
        lcol = lax.broadcasted_iota(jnp.int32, (_TILE, _NA), 1)
        amax = jnp.min(jnp.where(logits == m, lcol, _NA),
                       axis=1, keepdims=True)
        se = jnp.sum(jnp.exp(logits - m), axis=1, keepdims=True)
        lp = -jnp.log(se)
        flag = instr_ref[...] == e
        if e == 0:
            v_ref[...] = jnp.where(flag, v, 0.0)
            a_ref[...] = jnp.where(flag, amax, 0)
            lp_ref[...] = jnp.where(flag, lp, 0.0)
            h_ref[...] = jnp.where(flag, h, 0.0)
        else:
            v_ref[...] = jnp.where(flag, v, v_ref[...])
            a_ref[...] = jnp.where(flag, amax, a_ref[...])
            lp_ref[...] = jnp.where(flag, lp, lp_ref[...])
            h_ref[...] = jnp.where(flag, h, h_ref[...])


def kernel(base, instructions, rnn_hxs, masks, W1, b1, Wh, Wv, bv, Wa, ba):
    n, d = base.shape
    instr2 = instructions.reshape(n, 1)

    grid = (n // _TILE,)
    value, action, alp, h_out = pl.pallas_call(
        _tile_body,
        grid=grid,
        in_specs=[
            pl.BlockSpec((_TILE, 1), lambda t: (t, 0)),
            pl.BlockSpec((_TILE, 1), lambda t: (t, 0)),
            pl.BlockSpec((_TILE, d), lambda t: (t, 0)),
            pl.BlockSpec((_TILE, d), lambda t: (t, 0)),
            pl.BlockSpec((_N_EXP, d, d), lambda t: (0, 0, 0)),
            pl.BlockSpec((_N_EXP, d, d), lambda t: (0, 0, 0)),
            pl.BlockSpec((_NE_TOT, d), lambda t: (0, 0)),
            pl.BlockSpec((_N_EXP, d, _NA), lambda t: (0, 0, 0)),
            pl.BlockSpec((_NE_TOT, _NA), lambda t: (0, 0)),
            pl.BlockSpec((_N_EXP, d, 1), lambda t: (0, 0, 0)),
            pl.BlockSpec((_NE_TOT, 1), lambda t: (0, 0)),
        ],
        out_specs=[
            pl.BlockSpec((_TILE, 1), lambda t: (t, 0)),
            pl.BlockSpec((_TILE, 1), lambda t: (t, 0)),
            pl.BlockSpec((_TILE, 1), lambda t: (t, 0)),
            pl.BlockSpec((_TILE, d), lambda t: (t, 0)),
        ],
        out_shape=[
            jax.ShapeDtypeStruct((n, 1), jnp.float32),
            jax.ShapeDtypeStruct((n, 1), jnp.int32),
            jax.ShapeDtypeStruct((n, 1), jnp.float32),
            jax.ShapeDtypeStruct((n, d), jnp.float32),
        ],
        compiler_params=pltpu.CompilerParams(
            dimension_semantics=("arbitrary",),
        ),
    )(instr2, masks, base, rnn_hxs, W1, Wh, b1, Wa, ba, Wv, bv)

    return value, action, alp, h_out


# speedup vs baseline: 1.4630x; 1.0290x over previous
import jax
import jax.numpy as jnp
from jax import lax
from jax.experimental import pallas as pl
from jax.experimental.pallas import tpu as pltpu

_N_EXP = 2
_NE_TOT = 3
_D = 1024
_NA = 16
_TILE = 512


def _tile_body(instr_ref, mask_ref, x_ref, r_ref, w1_ref, wh_ref, b1_ref,
               wa_ref, ba_ref, wv_ref, bv_ref,
               v_ref, a_ref, lp_ref, h_ref):
    x = x_ref[...]
    r = r_ref[...] * mask_ref[...]
    for e in range(_N_EXP):
        pre = (jnp.dot(x, w1_ref[e], preferred_element_type=jnp.float32)
               + jnp.dot(r, wh_ref[e], preferred_element_type=jnp.float32)
               + b1_ref[e][None, :])
        h = jnp.tanh(pre)
        logits = (jnp.dot(h, wa_ref[e], preferred_element_type=jnp.float32)
                  + ba_ref[e][None, :])
        v = (jnp.dot(h, wv_ref[e], preferred_element_type=jnp.float32)
             + bv_ref[e][None, :])
        m = jnp.max(logits, axis=1, keepdims=True)
        lcol = lax.broadcasted_iota(jnp.int32, (_TILE, _NA), 1)
        amax = jnp.min(jnp.where(logits == m, lcol, _NA),
                       axis=1, keepdims=True)
        se = jnp.sum(jnp.exp(logits - m), axis=1, keepdims=True)
        lp = -jnp.log(se)
        flag = instr_ref[...] == e
        if e == 0:
            v_ref[...] = jnp.where(flag, v, 0.0)
            a_ref[...] = jnp.where(flag, amax, 0)
            lp_ref[...] = jnp.where(flag, lp, 0.0)
            h_ref[...] = jnp.where(flag, h, 0.0)
        else:
            v_ref[...] = jnp.where(flag, v, v_ref[...])
            a_ref[...] = jnp.where(flag, amax, a_ref[...])
            lp_ref[...] = jnp.where(flag, lp, lp_ref[...])
            h_ref[...] = jnp.where(flag, h, h_ref[...])


def kernel(base, instructions, rnn_hxs, masks, W1, b1, Wh, Wv, bv, Wa, ba):
    n, d = base.shape
    instr2 = instructions.reshape(n, 1)

    grid = (n // _TILE,)
    value, action, alp, h_out = pl.pallas_call(
        _tile_body,
        grid=grid,
        in_specs=[
            pl.BlockSpec((_TILE, 1), lambda t: (t, 0)),
            pl.BlockSpec((_TILE, 1), lambda t: (t, 0)),
            pl.BlockSpec((_TILE, d), lambda t: (t, 0)),
            pl.BlockSpec((_TILE, d), lambda t: (t, 0)),
            pl.BlockSpec((_N_EXP, d, d), lambda t: (0, 0, 0)),
            pl.BlockSpec((_N_EXP, d, d), lambda t: (0, 0, 0)),
            pl.BlockSpec((_NE_TOT, d), lambda t: (0, 0)),
            pl.BlockSpec((_N_EXP, d, _NA), lambda t: (0, 0, 0)),
            pl.BlockSpec((_NE_TOT, _NA), lambda t: (0, 0)),
            pl.BlockSpec((_N_EXP, d, 1), lambda t: (0, 0, 0)),
            pl.BlockSpec((_NE_TOT, 1), lambda t: (0, 0)),
        ],
        out_specs=[
            pl.BlockSpec((_TILE, 1), lambda t: (t, 0)),
            pl.BlockSpec((_TILE, 1), lambda t: (t, 0)),
            pl.BlockSpec((_TILE, 1), lambda t: (t, 0)),
            pl.BlockSpec((_TILE, d), lambda t: (t, 0)),
        ],
        out_shape=[
            jax.ShapeDtypeStruct((n, 1), jnp.float32),
            jax.ShapeDtypeStruct((n, 1), jnp.int32),
            jax.ShapeDtypeStruct((n, 1), jnp.float32),
            jax.ShapeDtypeStruct((n, d), jnp.float32),
        ],
        compiler_params=pltpu.CompilerParams(
            dimension_semantics=("arbitrary",),
        ),
    )(instr2, masks, base, rnn_hxs, W1, Wh, b1, Wa, ba, Wv, bv)

    return value, action, alp, h_out
